# Initial kernel scaffold; baseline (speedup 1.0000x reference)
import jax
import jax.numpy as jnp
from jax.experimental import pallas as pl


def kernel(hidden_states, gate_weight, w_gate, w_up, w_down, ws_gate, ws_up, ws_down):
    raise NotImplementedError("write your pallas kernel here")


# speedup vs baseline: 13.8920x; 13.8920x over previous
import jax
import jax.numpy as jnp
from jax.experimental import pallas as pl
from jax.experimental.pallas import tpu as pltpu

_E = 64
_TOPK = 8
_NEG = -1e30


def _silu(x):
    return x * jax.nn.sigmoid(x)


def _gate_kernel(h_ref, gw_ref, wdense_ref):
    h = h_ref[...]
    gw = gw_ref[...]
    logits = jax.lax.dot_general(
        h, gw, (((1,), (1,)), ((), ())), preferred_element_type=jnp.float32
    )
    iota_e = jax.lax.broadcasted_iota(jnp.int32, logits.shape, 1)

    s = logits
    sel = jnp.zeros_like(logits, dtype=jnp.bool_)
    for _ in range(_TOPK):
        m = jnp.max(s, axis=-1, keepdims=True)
        amax = jnp.min(jnp.where(s == m, iota_e, _E), axis=-1, keepdims=True)
        hit = iota_e == amax
        sel = jnp.logical_or(sel, hit)
        s = jnp.where(hit, _NEG, s)

    masked = jnp.where(sel, logits, _NEG)
    mx = jnp.max(masked, axis=-1, keepdims=True)
    ex = jnp.where(sel, jnp.exp(masked - mx), 0.0)
    w = ex / jnp.sum(ex, axis=-1, keepdims=True)
    wdense_ref[...] = w


def _dense_expert_kernel(wdense_ref, h_ref, wg_ref, wu_ref, wd_ref, acc_ref):
    e = pl.program_id(0)
    x = h_ref[...]
    g = jnp.dot(x, wg_ref[0], preferred_element_type=jnp.float32)
    u = jnp.dot(x, wu_ref[0], preferred_element_type=jnp.float32)
    y = jnp.dot(_silu(g) * u, wd_ref[0], preferred_element_type=jnp.float32)

    iota_e = jax.lax.broadcasted_iota(jnp.int32, wdense_ref.shape, 1)
    col = jnp.sum(
        jnp.where(iota_e == e, wdense_ref[...], 0.0), axis=1, keepdims=True
    )
    contrib = y * col

    @pl.when(e == 0)
    def _():
        acc_ref[...] = contrib

    @pl.when(e > 0)
    def _():
        acc_ref[...] += contrib


def _shared_final_kernel(h_ref, wsg_ref, wsu_ref, wsd_ref, acc_ref, out_ref):
    x = h_ref[...]
    g = jnp.dot(x, wsg_ref[...], preferred_element_type=jnp.float32)
    u = jnp.dot(x, wsu_ref[...], preferred_element_type=jnp.float32)
    y = jnp.dot(_silu(g) * u, wsd_ref[...], preferred_element_type=jnp.float32)
    out_ref[...] = y + acc_ref[...]


def kernel(hidden_states, gate_weight, w_gate, w_up, w_down, ws_gate, ws_up, ws_down):
    orig_shape = hidden_states.shape
    D = orig_shape[-1]
    h = hidden_states.reshape(-1, D)
    T = h.shape[0]
    E, _, F = w_gate.shape

    wdense = pl.pallas_call(
        _gate_kernel,
        out_shape=jax.ShapeDtypeStruct((T, E), jnp.float32),
    )(h, gate_weight)

    acc = pl.pallas_call(
        _dense_expert_kernel,
        grid=(E,),
        in_specs=[
            pl.BlockSpec((T, E), lambda e: (0, 0)),
            pl.BlockSpec((T, D), lambda e: (0, 0)),
            pl.BlockSpec((1, D, F), lambda e: (e, 0, 0)),
            pl.BlockSpec((1, D, F), lambda e: (e, 0, 0)),
            pl.BlockSpec((1, F, D), lambda e: (e, 0, 0)),
        ],
        out_specs=pl.BlockSpec((T, D), lambda e: (0, 0)),
        out_shape=jax.ShapeDtypeStruct((T, D), jnp.float32),
    )(wdense, h, w_gate, w_up, w_down)

    TT = 512
    out = pl.pallas_call(
        _shared_final_kernel,
        grid=(T // TT,),
        in_specs=[
            pl.BlockSpec((TT, D), lambda t: (t, 0)),
            pl.BlockSpec(ws_gate.shape, lambda t: (0, 0)),
            pl.BlockSpec(ws_up.shape, lambda t: (0, 0)),
            pl.BlockSpec(ws_down.shape, lambda t: (0, 0)),
            pl.BlockSpec((TT, D), lambda t: (t, 0)),
        ],
        out_specs=pl.BlockSpec((TT, D), lambda t: (t, 0)),
        out_shape=jax.ShapeDtypeStruct((T, D), jnp.float32),
    )(h, ws_gate, ws_up, ws_down, acc)

    return out.reshape(orig_shape)
